# baseline (speedup 1.0000x reference)
import jax
import jax.numpy as jnp
from jax import lax
from jax.experimental import pallas as pl
from jax.experimental.pallas import tpu as pltpu
from jax.experimental.pallas import tpu_sc as plsc

VOCAB = 1000
PADV = VOCAB
B, T = 1024, 200
NW = 32
CH, CL = 125, 8
BH, BL = 8, 128
R = 16
NPIECE = BL // R
NPAIR = (T * BH) // NW
NSTEP = NPAIR * NPIECE
NGB = 3
NST = 2


def _body(table_hbm, idx_hbm, out_hbm, idx_v, gbuf0, gbuf1, gbuf2,
          stg0, stg1, gsem0, gsem1, gsem2, osem0, osem1):
    gbufs = (gbuf0, gbuf1, gbuf2)
    stgs = (stg0, stg1)
    gsems = (gsem0, gsem1, gsem2)
    osems = (osem0, osem1)
    wid = lax.axis_index("s") * 2 + lax.axis_index("c")
    pair0 = wid * NPAIR

    pltpu.sync_copy(idx_hbm.at[pl.ds(pair0, NPAIR)], idx_v)

    lane = lax.iota(jnp.int32, 16)

    def gather_start(s, b):
        p, h = s // NPIECE, s % NPIECE
        pltpu.async_copy(
            table_hbm.at[idx_v.at[p, pl.ds(h * R, R)]], gbufs[b], gsems[b])

    def gather_wait(s, b):
        p, h = s // NPIECE, s % NPIECE
        pltpu.make_async_copy(
            table_hbm.at[idx_v.at[p, pl.ds(h * R, R)]], gbufs[b],
            gsems[b]).wait()

    def out_slice(s):
        p, h = s // NPIECE, s % NPIECE
        gp = pair0 + p
        t, b_hi = gp // BH, gp % BH
        return out_hbm.at[t, :, b_hi, :, pl.ds(h * R, R)]

    def write_start(s, b):
        pltpu.async_copy(stgs[b], out_slice(s), osems[b])

    def write_wait(s, b):
        pltpu.make_async_copy(stgs[b], out_slice(s), osems[b]).wait()

    def transpose(gb, st):
        gbuf, stg = gbufs[gb], stgs[st]

        def step(c_hi, wrap):
            wbase = jnp.full((16,), c_hi * CL, jnp.int32) + lane
            for c_lo in range(CL):
                w = wbase + c_lo
                col = jnp.where(w >= VOCAB, w - VOCAB, w) if wrap else w
                v = plsc.load_gather(gbuf, [lane, col])
                plsc.store_scatter(
                    stg, [col >> 3, col & 7, lane], v)

        @pl.loop(0, CH - 2, unroll=4)
        def _(c_hi):
            step(c_hi, wrap=False)

        for c_hi in (CH - 2, CH - 1):
            step(c_hi, wrap=True)

    for s0 in range(NGB):
        gather_start(s0, s0)

    @pl.loop(0, NSTEP // 6)
    def _(k):
        for j in range(6):
            s = k * 6 + j
            gather_wait(s, j % NGB)
            if j < NST:
                @pl.when(k >= 1)
                def _():
                    write_wait(s - NST, j % NST)
            else:
                write_wait(s - NST, j % NST)
            transpose(j % NGB, j % NST)
            gather_start(s + NGB, j % NGB)
            write_start(s, j % NST)

    for s in range(NSTEP - 4, NSTEP):
        gather_wait(s, s % NGB)
        write_wait(s - NST, s % NST)
        transpose(s % NGB, s % NST)
        if s + NGB < NSTEP:
            gather_start(s + NGB, s % NGB)
        write_start(s, s % NST)

    for s in (NSTEP - NST, NSTEP - 1):
        write_wait(s, s % NST)


def kernel(inps, table):
    idx = inps.T.reshape(T * BH, BL)
    table_pad = table
    mesh = plsc.VectorSubcoreMesh(core_axis_name="c", subcore_axis_name="s")
    run = pl.kernel(
        _body,
        out_type=jax.ShapeDtypeStruct((T, CH, BH, CL, BL), jnp.float32),
        mesh=mesh,
        scratch_types=[
            pltpu.VMEM((NPAIR, BL), jnp.int32),
            pltpu.VMEM((R, PADV), jnp.float32),
            pltpu.VMEM((R, PADV), jnp.float32),
            pltpu.VMEM((R, PADV), jnp.float32),
            pltpu.VMEM((CH, CL, R), jnp.float32),
            pltpu.VMEM((CH, CL, R), jnp.float32),
            pltpu.SemaphoreType.DMA,
            pltpu.SemaphoreType.DMA,
            pltpu.SemaphoreType.DMA,
            pltpu.SemaphoreType.DMA,
            pltpu.SemaphoreType.DMA,
        ],
        compiler_params=pltpu.CompilerParams(use_tc_tiling_on_sc=False,
                                             needs_layout_passes=False),
    )
    out5 = run(table_pad, idx)
    return out5.transpose(2, 4, 0, 1, 3).reshape(B, T, VOCAB)


# speedup vs baseline: 1.0043x; 1.0043x over previous
import jax
import jax.numpy as jnp
from jax import lax
from jax.experimental import pallas as pl
from jax.experimental.pallas import tpu as pltpu
from jax.experimental.pallas import tpu_sc as plsc

VOCAB = 1000
B, T = 1024, 200
NW = 32
CH, CL = 125, 8
BH, BL = 8, 128
R = 16
NPIECE = BL // R
NPAIR = (T * BH) // NW
NSTEP = NPAIR * NPIECE
NBUF = 2


def _body(table_hbm, idx_hbm, out_hbm, idx_v, gbuf0, gbuf1,
          stg0, stg1, gsem0, gsem1, osem0, osem1):
    gbufs = (gbuf0, gbuf1)
    stgs = (stg0, stg1)
    gsems = (gsem0, gsem1)
    osems = (osem0, osem1)
    wid = lax.axis_index("s") * 2 + lax.axis_index("c")
    pair0 = wid * NPAIR

    pltpu.sync_copy(idx_hbm.at[pl.ds(pair0, NPAIR)], idx_v)

    lane = lax.iota(jnp.int32, 16)

    def gather_start(s, b):
        p, h = s // NPIECE, s % NPIECE
        pltpu.async_copy(
            table_hbm.at[idx_v.at[p, pl.ds(h * R, R)]], gbufs[b], gsems[b])

    def gather_wait(s, b):
        p, h = s // NPIECE, s % NPIECE
        pltpu.make_async_copy(
            table_hbm.at[idx_v.at[p, pl.ds(h * R, R)]], gbufs[b],
            gsems[b]).wait()

    def out_slice(s):
        p, h = s // NPIECE, s % NPIECE
        gp = pair0 + p
        t, b_hi = gp // BH, gp % BH
        return out_hbm.at[t, :, b_hi, :, pl.ds(h * R, R)]

    def write_start(s, b):
        pltpu.async_copy(stgs[b], out_slice(s), osems[b])

    def write_wait(s, b):
        pltpu.make_async_copy(stgs[b], out_slice(s), osems[b]).wait()

    def transpose(gb, st):
        gbuf, stg = gbufs[gb], stgs[st]

        def step(c_hi, wrap):
            wbase = jnp.full((16,), c_hi * CL, jnp.int32) + lane
            for c_lo in range(CL):
                w = wbase + c_lo
                col = jnp.where(w >= VOCAB, w - VOCAB, w) if wrap else w
                v = plsc.load_gather(gbuf, [lane, col])
                plsc.store_scatter(
                    stg, [col >> 3, col & 7, lane], v)

        @pl.loop(0, CH - 2, unroll=4)
        def _(c_hi):
            step(c_hi, wrap=False)

        for c_hi in (CH - 2, CH - 1):
            step(c_hi, wrap=True)

    for s0 in range(NBUF):
        gather_start(s0, s0)

    @pl.loop(0, NSTEP // NBUF)
    def _(k):
        for b in range(NBUF):
            s = k * NBUF + b
            gather_wait(s, b)

            @pl.when(k >= 1)
            def _():
                write_wait(s - NBUF, b)

            transpose(b, b)

            @pl.when(k < NSTEP // NBUF - 1)
            def _():
                gather_start(s + NBUF, b)

            write_start(s, b)

    for b in range(NBUF):
        write_wait(NSTEP - NBUF + b, b)


def kernel(inps, table):
    idx = inps.T.reshape(T * BH, BL)
    mesh = plsc.VectorSubcoreMesh(core_axis_name="c", subcore_axis_name="s")
    run = pl.kernel(
        _body,
        out_type=jax.ShapeDtypeStruct((T, CH, BH, CL, BL), jnp.float32),
        mesh=mesh,
        scratch_types=[
            pltpu.VMEM((NPAIR, BL), jnp.int32),
            pltpu.VMEM((R, VOCAB), jnp.float32),
            pltpu.VMEM((R, VOCAB), jnp.float32),
            pltpu.VMEM((CH, CL, R), jnp.float32),
            pltpu.VMEM((CH, CL, R), jnp.float32),
            pltpu.SemaphoreType.DMA,
            pltpu.SemaphoreType.DMA,
            pltpu.SemaphoreType.DMA,
            pltpu.SemaphoreType.DMA,
        ],
        compiler_params=pltpu.CompilerParams(use_tc_tiling_on_sc=False,
                                             needs_layout_passes=False),
    )
    out5 = run(table, idx)
    return out5.transpose(2, 4, 0, 1, 3).reshape(B, T, VOCAB)
